# baseline (speedup 1.0000x reference)
import functools

import jax
import jax.numpy as jnp
from jax import lax
from jax.experimental import pallas as pl
from jax.experimental.pallas import tpu as pltpu
from jax.experimental.pallas import tpu_sc as plsc

D_NODE = 128
D_EDGE = 16
D_OUT = D_NODE + D_EDGE
BLK = 128
N_WORKERS = 32


def _node_proj_kernel(x_ref, wt_ref, b_ref, o_ref):
    o_ref[...] = (
        jnp.dot(x_ref[...], wt_ref[...], preferred_element_type=jnp.float32)
        + b_ref[...]
    )


def _node_project(x, wt, b, block_rows):
    n = x.shape[0]
    d = x.shape[1]
    return pl.pallas_call(
        _node_proj_kernel,
        grid=(n // block_rows,),
        in_specs=[
            pl.BlockSpec((block_rows, d), lambda i: (i, 0)),
            pl.BlockSpec((d, d), lambda i: (0, 0)),
            pl.BlockSpec((1, d), lambda i: (0, 0)),
        ],
        out_specs=pl.BlockSpec((block_rows, d), lambda i: (i, 0)),
        out_shape=jax.ShapeDtypeStruct((n, d), jnp.float32),
    )(x, wt, b.reshape(1, d))


def _edge_proj_kernel(ef_ref, w_ref, b_ref, alias_ref, o_ref):
    del alias_ref
    o_ref[...] = (
        jnp.dot(w_ref[...], ef_ref[...], preferred_element_type=jnp.float32)
        + b_ref[...]
    )


def _edge_project_into(ef_t, w, b, out_t, block_cols):
    e = ef_t.shape[1]
    grid = e // block_cols
    return pl.pallas_call(
        _edge_proj_kernel,
        grid=(grid,),
        in_specs=[
            pl.BlockSpec((D_EDGE, block_cols), lambda j: (0, j)),
            pl.BlockSpec((D_EDGE, D_EDGE), lambda j: (0, 0)),
            pl.BlockSpec((D_EDGE, 1), lambda j: (0, 0)),
            pl.BlockSpec(memory_space=pl.ANY),
        ],
        out_specs=pl.BlockSpec((D_EDGE, block_cols), lambda j: (8, j)),
        out_shape=jax.ShapeDtypeStruct((D_OUT, e), jnp.float32),
        input_output_aliases={3: 0},
    )(ef_t, w, b.reshape(D_EDGE, 1), out_t)


def _make_sc_gather_t(n_edges):
    n_blocks = n_edges // BLK
    mesh = plsc.VectorSubcoreMesh(core_axis_name="c", subcore_axis_name="s")

    @functools.partial(
        pl.kernel,
        mesh=mesh,
        compiler_params=pltpu.CompilerParams(
            use_tc_tiling_on_sc=True, needs_layout_passes=False
        ),
        out_type=jax.ShapeDtypeStruct((D_OUT, n_edges), jnp.float32),
        scratch_types=[
            pltpu.VMEM((BLK,), jnp.int32),
            pltpu.VMEM((BLK, D_NODE), jnp.float32),
            pltpu.VMEM((16, 8, BLK), jnp.float32),
            pltpu.SemaphoreType.DMA,
        ],
    )
    def sc_gather_t(node_h_hbm, src_hbm, out_hbm, idx_v, rows_v, tbuf_v, sem):
        wid = lax.axis_index("s") * 2 + lax.axis_index("c")
        lo = (wid * n_blocks) // N_WORKERS
        hi = ((wid + 1) * n_blocks) // N_WORKERS

        def body(j, carry):
            e0 = j * BLK
            pltpu.sync_copy(src_hbm.at[pl.ds(e0, BLK)], idx_v)
            pltpu.async_copy(node_h_hbm.at[idx_v], rows_v, sem).wait()

            def trans(i, carry2):
                for r in range(8):
                    dvec = jnp.full((16,), 8 * i + r, jnp.int32)
                    for v in range(8):
                        evec = 16 * v + lax.iota(jnp.int32, 16)
                        vals = plsc.load_gather(rows_v, [evec, dvec])
                        tbuf_v[i, r, pl.ds(16 * v, 16)] = vals
                return carry2

            lax.fori_loop(0, 16, trans, 0)

            for i in range(16):
                pltpu.sync_copy(
                    tbuf_v.at[i], out_hbm.at[pl.ds(8 * i, 8), pl.ds(e0, BLK)]
                )
            return carry

        lax.fori_loop(lo, hi, body, 0)

    return sc_gather_t


def kernel(node_feats, edge_feats, W_node, b_node, W_edge, b_edge, edge_index):
    n_edges = edge_feats.shape[0]

    node_h = _node_project(node_feats, W_node.T, b_node, block_rows=2000)
    src = jnp.asarray(edge_index[0], dtype=jnp.int32)

    out_t = _make_sc_gather_t(n_edges)(node_h, src)
    out_t = _edge_project_into(edge_feats.T, W_edge, b_edge, out_t, block_cols=6400)
    return out_t.T


# speedup vs baseline: 5.8533x; 5.8533x over previous
import functools

import jax
import jax.numpy as jnp
from jax import lax
from jax.experimental import pallas as pl
from jax.experimental.pallas import tpu as pltpu
from jax.experimental.pallas import tpu_sc as plsc

D_NODE = 128
D_EDGE = 16
D_OUT = D_NODE + D_EDGE
BLK = 128
N_WORKERS = 32
MAX_BLOCKS_PER_W = 79


def _node_proj_kernel(x_ref, wt_ref, b_ref, o_ref):
    o_ref[...] = (
        jnp.dot(x_ref[...], wt_ref[...], preferred_element_type=jnp.float32)
        + b_ref[...]
    )


def _node_project(x, wt, b, block_rows):
    n = x.shape[0]
    d = x.shape[1]
    return pl.pallas_call(
        _node_proj_kernel,
        grid=(n // block_rows,),
        in_specs=[
            pl.BlockSpec((block_rows, d), lambda i: (i, 0)),
            pl.BlockSpec((d, d), lambda i: (0, 0)),
            pl.BlockSpec((1, d), lambda i: (0, 0)),
        ],
        out_specs=pl.BlockSpec((block_rows, d), lambda i: (i, 0)),
        out_shape=jax.ShapeDtypeStruct((n, d), jnp.float32),
    )(x, wt, b.reshape(1, d))


def _edge_proj_kernel(ef_ref, w_ref, b_ref, alias_ref, o_ref):
    del alias_ref
    o_ref[...] = (
        jnp.dot(w_ref[...], ef_ref[...], preferred_element_type=jnp.float32)
        + b_ref[...]
    )


def _edge_project_into(ef_t, w, b, out_t, block_cols):
    e = ef_t.shape[1]
    grid = e // block_cols
    return pl.pallas_call(
        _edge_proj_kernel,
        grid=(grid,),
        in_specs=[
            pl.BlockSpec((D_EDGE, block_cols), lambda j: (0, j)),
            pl.BlockSpec((D_EDGE, D_EDGE), lambda j: (0, 0)),
            pl.BlockSpec((D_EDGE, 1), lambda j: (0, 0)),
            pl.BlockSpec(memory_space=pl.ANY),
        ],
        out_specs=pl.BlockSpec((D_EDGE, block_cols), lambda j: (8, j)),
        out_shape=jax.ShapeDtypeStruct((D_OUT, e), jnp.float32),
        input_output_aliases={3: 0},
    )(ef_t, w, b.reshape(D_EDGE, 1), out_t)


def _make_sc_gather_t(n_edges):
    n_blocks = n_edges // BLK
    idx_cap = MAX_BLOCKS_PER_W * BLK
    mesh = plsc.VectorSubcoreMesh(core_axis_name="c", subcore_axis_name="s")

    @functools.partial(
        pl.kernel,
        mesh=mesh,
        compiler_params=pltpu.CompilerParams(
            use_tc_tiling_on_sc=True, needs_layout_passes=False
        ),
        out_type=jax.ShapeDtypeStruct((D_OUT, n_edges), jnp.float32),
        scratch_types=[
            pltpu.VMEM((idx_cap,), jnp.int32),
            pltpu.VMEM((BLK, D_NODE), jnp.float32),
            pltpu.VMEM((BLK, D_NODE), jnp.float32),
            pltpu.VMEM((D_NODE, BLK), jnp.float32),
            pltpu.VMEM((D_NODE, BLK), jnp.float32),
            pltpu.SemaphoreType.DMA,
            pltpu.SemaphoreType.DMA,
            pltpu.SemaphoreType.DMA,
            pltpu.SemaphoreType.DMA,
        ],
    )
    def sc_gather_t(
        node_h_hbm,
        src_hbm,
        out_hbm,
        idx_all,
        rows_a,
        rows_b,
        tbuf_a,
        tbuf_b,
        sem_ga,
        sem_gb,
        sem_oa,
        sem_ob,
    ):
        wid = lax.axis_index("s") * 2 + lax.axis_index("c")
        lo = (wid * n_blocks) // N_WORKERS
        hi = ((wid + 1) * n_blocks) // N_WORKERS
        nb = hi - lo

        pltpu.sync_copy(src_hbm.at[pl.ds(lo * BLK, idx_cap)], idx_all)

        def launch_gather(j, rows_x, sem_x):
            idx_slice = idx_all.at[pl.ds((j - lo) * BLK, BLK)]
            pltpu.async_copy(node_h_hbm.at[idx_slice], rows_x, sem_x)

        def wait_gather(rows_x, sem_x):
            pltpu.make_async_copy(node_h_hbm.at[idx_all.at[pl.ds(0, BLK)]], rows_x, sem_x).wait()

        def out_window(j):
            return out_hbm.at[pl.ds(0, D_NODE), pl.ds(j * BLK, BLK)]

        def transpose(rows_x, tbuf_x):
            @functools.partial(plsc.parallel_loop, 0, 16, unroll=2)
            def _(i):
                for r in range(8):
                    d = 8 * i + r
                    dvec = jnp.full((16,), d, jnp.int32)
                    for v in range(8):
                        evec = 16 * v + lax.iota(jnp.int32, 16)
                        tbuf_x[d, pl.ds(16 * v, 16)] = plsc.load_gather(
                            rows_x, [evec, dvec]
                        )

        def step(j, rows_x, sem_gx, tbuf_x, sem_ox, rows_y, sem_gy):
            @pl.when(j < hi)
            def _():
                wait_gather(rows_x, sem_gx)

                @pl.when(j + 1 < hi)
                def _():
                    launch_gather(j + 1, rows_y, sem_gy)

                @pl.when(j >= lo + 2)
                def _():
                    pltpu.make_async_copy(tbuf_x, out_window(j), sem_ox).wait()

                transpose(rows_x, tbuf_x)
                pltpu.async_copy(tbuf_x, out_window(j), sem_ox)

        launch_gather(lo, rows_a, sem_ga)

        def pair(jj, carry):
            j0 = lo + 2 * jj
            step(j0, rows_a, sem_ga, tbuf_a, sem_oa, rows_b, sem_gb)
            step(j0 + 1, rows_b, sem_gb, tbuf_b, sem_ob, rows_a, sem_ga)
            return carry

        lax.fori_loop(0, (nb + 1) // 2, pair, 0)

        @pl.when(nb >= 1)
        def _():
            pltpu.make_async_copy(tbuf_a, out_window(lo), sem_oa).wait()

        @pl.when(nb >= 2)
        def _():
            pltpu.make_async_copy(tbuf_b, out_window(lo), sem_ob).wait()

    return sc_gather_t


def kernel(node_feats, edge_feats, W_node, b_node, W_edge, b_edge, edge_index):
    n_edges = edge_feats.shape[0]

    node_h = _node_project(node_feats, W_node.T, b_node, block_rows=2000)
    src = jnp.asarray(edge_index[0], dtype=jnp.int32)

    out_t = _make_sc_gather_t(n_edges)(node_h, src)
    out_t = _edge_project_into(edge_feats.T, W_edge, b_edge, out_t, block_cols=6400)
    return out_t.T


# speedup vs baseline: 6.7682x; 1.1563x over previous
import functools

import jax
import jax.numpy as jnp
from jax import lax
from jax.experimental import pallas as pl
from jax.experimental.pallas import tpu as pltpu
from jax.experimental.pallas import tpu_sc as plsc

D_NODE = 128
D_EDGE = 16
D_OUT = D_NODE + D_EDGE
BLK = 128
N_WORKERS = 32
MAX_BLOCKS_PER_W = 79


def _node_proj_kernel(x_ref, wt_ref, b_ref, o_ref):
    o_ref[...] = (
        jnp.dot(x_ref[...], wt_ref[...], preferred_element_type=jnp.float32)
        + b_ref[...]
    )


def _node_project(x, wt, b, block_rows):
    n = x.shape[0]
    d = x.shape[1]
    return pl.pallas_call(
        _node_proj_kernel,
        grid=(n // block_rows,),
        in_specs=[
            pl.BlockSpec((block_rows, d), lambda i: (i, 0)),
            pl.BlockSpec((d, d), lambda i: (0, 0)),
            pl.BlockSpec((1, d), lambda i: (0, 0)),
        ],
        out_specs=pl.BlockSpec((block_rows, d), lambda i: (i, 0)),
        out_shape=jax.ShapeDtypeStruct((n, d), jnp.float32),
    )(x, wt, b.reshape(1, d))


def _edge_proj_kernel(ef_ref, w_ref, b_ref, alias_ref, o_ref):
    del alias_ref
    o_ref[...] = (
        jnp.dot(w_ref[...], ef_ref[...], preferred_element_type=jnp.float32)
        + b_ref[...]
    )


def _edge_project_into(ef_t, w, b, out_t, block_cols):
    e = ef_t.shape[1]
    grid = e // block_cols
    return pl.pallas_call(
        _edge_proj_kernel,
        grid=(grid,),
        in_specs=[
            pl.BlockSpec((D_EDGE, block_cols), lambda j: (0, j)),
            pl.BlockSpec((D_EDGE, D_EDGE), lambda j: (0, 0)),
            pl.BlockSpec((D_EDGE, 1), lambda j: (0, 0)),
            pl.BlockSpec(memory_space=pl.ANY),
        ],
        out_specs=pl.BlockSpec((D_EDGE, block_cols), lambda j: (8, j)),
        out_shape=jax.ShapeDtypeStruct((D_OUT, e), jnp.float32),
        input_output_aliases={3: 0},
    )(ef_t, w, b.reshape(D_EDGE, 1), out_t)


def _make_sc_gather_t(n_edges):
    n_blocks = n_edges // BLK
    idx_cap = MAX_BLOCKS_PER_W * BLK
    mesh = plsc.VectorSubcoreMesh(core_axis_name="c", subcore_axis_name="s")

    @functools.partial(
        pl.kernel,
        mesh=mesh,
        compiler_params=pltpu.CompilerParams(
            use_tc_tiling_on_sc=True, needs_layout_passes=False
        ),
        out_type=jax.ShapeDtypeStruct((D_OUT, n_edges), jnp.float32),
        scratch_types=[
            pltpu.VMEM((1, idx_cap), jnp.int32),
            pltpu.VMEM((BLK, D_NODE), jnp.float32),
            pltpu.VMEM((BLK, D_NODE), jnp.float32),
            pltpu.VMEM((D_NODE, BLK), jnp.float32),
            pltpu.VMEM((D_NODE, BLK), jnp.float32),
            pltpu.SemaphoreType.DMA,
            pltpu.SemaphoreType.DMA,
            pltpu.SemaphoreType.DMA,
            pltpu.SemaphoreType.DMA,
        ],
    )
    def sc_gather_t(
        node_h_hbm,
        src_hbm,
        out_hbm,
        idx_all,
        rows_a,
        rows_b,
        tbuf_a,
        tbuf_b,
        sem_ga,
        sem_gb,
        sem_oa,
        sem_ob,
    ):
        wid = lax.axis_index("s") * 2 + lax.axis_index("c")
        lo = (wid * n_blocks) // N_WORKERS
        hi = ((wid + 1) * n_blocks) // N_WORKERS
        nb = hi - lo

        pltpu.sync_copy(src_hbm.at[pl.ds(0, 1), pl.ds(lo * BLK, idx_cap)], idx_all)

        def launch_gather(j, rows_x, sem_x):
            idx_slice = idx_all.at[0, pl.ds((j - lo) * BLK, BLK)]
            pltpu.async_copy(node_h_hbm.at[idx_slice], rows_x, sem_x)

        def wait_gather(rows_x, sem_x):
            pltpu.make_async_copy(
                node_h_hbm.at[idx_all.at[0, pl.ds(0, BLK)]], rows_x, sem_x
            ).wait()

        def out_window(j):
            return out_hbm.at[pl.ds(0, D_NODE), pl.ds(j * BLK, BLK)]

        def transpose(rows_x, tbuf_x):
            @functools.partial(plsc.parallel_loop, 0, 16, unroll=2)
            def _(i):
                for r in range(8):
                    d = 8 * i + r
                    dvec = jnp.full((16,), d, jnp.int32)
                    for v in range(8):
                        evec = 16 * v + lax.iota(jnp.int32, 16)
                        tbuf_x[d, pl.ds(16 * v, 16)] = plsc.load_gather(
                            rows_x, [evec, dvec]
                        )

        def step(j, rows_x, sem_gx, tbuf_x, sem_ox, rows_y, sem_gy):
            @pl.when(j < hi)
            def _():
                wait_gather(rows_x, sem_gx)

                @pl.when(j + 1 < hi)
                def _():
                    launch_gather(j + 1, rows_y, sem_gy)

                @pl.when(j >= lo + 2)
                def _():
                    pltpu.make_async_copy(tbuf_x, out_window(j), sem_ox).wait()

                transpose(rows_x, tbuf_x)
                pltpu.async_copy(tbuf_x, out_window(j), sem_ox)

        launch_gather(lo, rows_a, sem_ga)

        def pair(jj, carry):
            j0 = lo + 2 * jj
            step(j0, rows_a, sem_ga, tbuf_a, sem_oa, rows_b, sem_gb)
            step(j0 + 1, rows_b, sem_gb, tbuf_b, sem_ob, rows_a, sem_ga)
            return carry

        lax.fori_loop(0, (nb + 1) // 2, pair, 0)

        @pl.when(nb >= 1)
        def _():
            pltpu.make_async_copy(tbuf_a, out_window(lo), sem_oa).wait()

        @pl.when(nb >= 2)
        def _():
            pltpu.make_async_copy(tbuf_b, out_window(lo), sem_ob).wait()

    return sc_gather_t


def kernel(node_feats, edge_feats, W_node, b_node, W_edge, b_edge, edge_index):
    n_edges = edge_feats.shape[0]

    node_h = _node_project(node_feats, W_node.T, b_node, block_rows=2000)
    src2d = jnp.asarray(edge_index, dtype=jnp.int32)

    out_t = _make_sc_gather_t(n_edges)(node_h, src2d)
    out_t = _edge_project_into(edge_feats.T, W_edge, b_edge, out_t, block_cols=16000)
    return out_t.T
